# baseline (speedup 1.0000x reference)
import functools

import jax
import jax.numpy as jnp
from jax import lax
from jax.experimental import pallas as pl
from jax.experimental.pallas import tpu as pltpu
from jax.experimental.pallas import tpu_sc as plsc

B, S = 256, 512
EMBED = 768
D4 = EMBED // 4
D2 = EMBED // 2
NF, NT = 8, 64
NFI, NTI = 2 * NF - 1, 2 * NT - 1
NPOS = B * S
NC, NS = 2, 16
NW = NC * NS
ROWS_PER_W = B // NW
CHUNK = 64
NCHUNK = S // CHUNK
L = 16


def _make_sc_kernel():
    mesh = plsc.VectorSubcoreMesh(core_axis_name="c", subcore_axis_name="s")

    @functools.partial(
        pl.kernel,
        mesh=mesh,
        out_type=jax.ShapeDtypeStruct((NPOS, EMBED), jnp.float32),
        compiler_params=pltpu.CompilerParams(needs_layout_passes=False),
        scratch_types=[
            pltpu.VMEM((S,), jnp.int32),
            pltpu.VMEM((S,), jnp.int32),
            pltpu.VMEM((NCHUNK, CHUNK), jnp.int32),
            pltpu.VMEM((CHUNK, EMBED), jnp.float32),
            pltpu.VMEM((2 * D2,), jnp.float32),
            pltpu.VMEM((D2,), jnp.float32),
            pltpu.VMEM((L,), jnp.float32),
            pltpu.SemaphoreType.DMA,
        ],
    )
    def k(fp_hbm, tp_hbm, full_hbm, w_hbm, out_hbm,
          fpb, tpb, jidx, stage, wbuf, rbuf, redb, semg):
        wid = lax.axis_index("s") * NC + lax.axis_index("c")
        pltpu.sync_copy(w_hbm, wbuf)
        lanes = lax.broadcasted_iota(jnp.int32, (L,), 0)

        def lane_total(v):
            for step in (1, 2, 4, 8):
                redb[...] = v
                v = v + plsc.load_gather(redb, [jnp.bitwise_xor(lanes, step)])
            return v

        def row_body(r, carry):
            base = (wid * ROWS_PER_W + r) * S
            pltpu.sync_copy(fp_hbm.at[pl.ds(base, S)], fpb)
            pltpu.sync_copy(tp_hbm.at[pl.ds(base, S)], tpb)
            facc = jnp.zeros((L,), jnp.float32)
            tacc = jnp.zeros((L,), jnp.float32)
            for g in range(S // L):
                facc = facc + fpb[pl.ds(L * g, L)].astype(jnp.float32)
                tacc = tacc + tpb[pl.ds(L * g, L)].astype(jnp.float32)
            fc = lane_total(facc) * (1.0 / S)
            tc = lane_total(tacc) * (1.0 / S)
            af = (NF - 1) - fc
            at = (NT - 1) - tc
            kf = af.astype(jnp.int32)
            kt = at.astype(jnp.int32)
            df = af - kf.astype(jnp.float32)
            dt = at - kt.astype(jnp.float32)
            off = kf * NTI + kt
            for g in range(S // L):
                fv = fpb[pl.ds(L * g, L)]
                tv = tpb[pl.ds(L * g, L)]
                jidx[g // (CHUNK // L), pl.ds((g % (CHUNK // L)) * L, L)] = (
                    fv * NTI + tv + off)
            for j in range(D2 // L):
                rbuf[pl.ds(L * j, L)] = (df * wbuf[pl.ds(L * j, L)]
                                         + dt * wbuf[pl.ds(D2 + L * j, L)])

            def chunk_body(c, carry2):
                pltpu.async_copy(full_hbm.at[jidx.at[c]], stage, semg).wait()
                rv = [rbuf[pl.ds(L * j, L)] for j in range(D2 // L)]
                for p in range(CHUNK):
                    for j in range(D2 // L):
                        plsc.addupdate(stage.at[p, pl.ds(L * j, L)], rv[j])
                pltpu.sync_copy(stage, out_hbm.at[pl.ds(base + c * CHUNK, CHUNK)])
                return carry2

            lax.fori_loop(0, NCHUNK, chunk_body, 0)
            return carry

        lax.fori_loop(0, ROWS_PER_W, row_body, 0)

    return k


_sc_call = _make_sc_kernel()


def kernel(freq_positions, time_positions, freq_relative_emb, time_relative_emb, W_dist, b_dist):
    fp = freq_positions.reshape(-1).astype(jnp.int32)
    tp = time_positions.reshape(-1).astype(jnp.int32)
    vf = jnp.arange(NFI, dtype=jnp.float32) - (NF - 1)
    vt = jnp.arange(NTI, dtype=jnp.float32) - (NT - 1)
    dist = (vf[:, None, None] * W_dist[0][None, None, :]
            + vt[None, :, None] * W_dist[1][None, None, :]
            + b_dist[None, None, :])
    fpart = jnp.broadcast_to(freq_relative_emb[:, None, :], (NFI, NTI, D4))
    tpart = jnp.broadcast_to(time_relative_emb[None, :, :], (NFI, NTI, D4))
    full = jnp.concatenate([dist, fpart, tpart], axis=-1).reshape(NFI * NTI, EMBED)
    wflat = jnp.concatenate([W_dist[0], W_dist[1]])
    out = _sc_call(fp, tp, full, wflat)
    return out.reshape(B, S, EMBED)


# speedup vs baseline: 1.2178x; 1.2178x over previous
import functools

import jax
import jax.numpy as jnp
from jax import lax
from jax.experimental import pallas as pl
from jax.experimental.pallas import tpu as pltpu
from jax.experimental.pallas import tpu_sc as plsc

B, S = 256, 512
EMBED = 768
D4 = EMBED // 4
D2 = EMBED // 2
NF, NT = 8, 64
NFI, NTI = 2 * NF - 1, 2 * NT - 1
NPOS = B * S
NC, NS = 2, 16
NW = NC * NS
ROWS_PER_W = B // NW
CHUNK = 64
NCHUNK = S // CHUNK
L = 16


def _make_sc_kernel():
    mesh = plsc.VectorSubcoreMesh(core_axis_name="c", subcore_axis_name="s")

    @functools.partial(
        pl.kernel,
        mesh=mesh,
        out_type=jax.ShapeDtypeStruct((NPOS, EMBED), jnp.float32),
        compiler_params=pltpu.CompilerParams(needs_layout_passes=False),
        scratch_types=[
            pltpu.VMEM((S,), jnp.int32),
            pltpu.VMEM((S,), jnp.int32),
            pltpu.VMEM((NCHUNK, CHUNK), jnp.int32),
            pltpu.VMEM((2, CHUNK, EMBED), jnp.float32),
            pltpu.VMEM((2 * D2,), jnp.float32),
            pltpu.VMEM((D2,), jnp.float32),
            pltpu.VMEM((L,), jnp.float32),
            pltpu.SemaphoreType.DMA,
            pltpu.SemaphoreType.DMA,
            pltpu.SemaphoreType.DMA,
            pltpu.SemaphoreType.DMA,
        ],
    )
    def k(fp_hbm, tp_hbm, full_hbm, w_hbm, out_hbm,
          fpb, tpb, jidx, stage, wbuf, rbuf, redb, gs0, gs1, ws0, ws1):
        wid = lax.axis_index("s") * NC + lax.axis_index("c")
        pltpu.sync_copy(w_hbm, wbuf)
        lanes = lax.broadcasted_iota(jnp.int32, (L,), 0)

        def lane_total(v):
            for step in (1, 2, 4, 8):
                redb[...] = v
                v = v + plsc.load_gather(redb, [jnp.bitwise_xor(lanes, step)])
            return v

        def row_body(r, carry):
            base = (wid * ROWS_PER_W + r) * S
            pltpu.sync_copy(fp_hbm.at[pl.ds(base, S)], fpb)
            pltpu.sync_copy(tp_hbm.at[pl.ds(base, S)], tpb)
            facc = jnp.zeros((L,), jnp.float32)
            tacc = jnp.zeros((L,), jnp.float32)
            for g in range(S // L):
                facc = facc + fpb[pl.ds(L * g, L)].astype(jnp.float32)
                tacc = tacc + tpb[pl.ds(L * g, L)].astype(jnp.float32)
            fc = lane_total(facc) * (1.0 / S)
            tc = lane_total(tacc) * (1.0 / S)
            af = (NF - 1) - fc
            at = (NT - 1) - tc
            kf = af.astype(jnp.int32)
            kt = at.astype(jnp.int32)
            df = af - kf.astype(jnp.float32)
            dt = at - kt.astype(jnp.float32)
            off = kf * NTI + kt
            for g in range(S // L):
                fv = fpb[pl.ds(L * g, L)]
                tv = tpb[pl.ds(L * g, L)]
                jidx[g // (CHUNK // L), pl.ds((g % (CHUNK // L)) * L, L)] = (
                    fv * NTI + tv + off)
            for j in range(D2 // L):
                rbuf[pl.ds(L * j, L)] = (df * wbuf[pl.ds(L * j, L)]
                                         + dt * wbuf[pl.ds(D2 + L * j, L)])

            rv = [rbuf[pl.ds(L * j, L)] for j in range(D2 // L)]

            def add_bias(buf):
                for p in range(CHUNK):
                    for j in range(D2 // L):
                        plsc.addupdate(stage.at[buf, p, pl.ds(L * j, L)], rv[j])

            def gather(c, buf, sem):
                pltpu.async_copy(full_hbm.at[jidx.at[c]], stage.at[buf], sem)

            def gather_wait(c, buf, sem):
                pltpu.make_async_copy(full_hbm.at[jidx.at[c]], stage.at[buf], sem).wait()

            def write(c, buf, sem):
                dst = out_hbm.at[pl.ds(base + c * CHUNK, CHUNK)]
                pltpu.async_copy(stage.at[buf], dst, sem)

            def write_wait(c, buf, sem):
                dst = out_hbm.at[pl.ds(base + c * CHUNK, CHUNK)]
                pltpu.make_async_copy(stage.at[buf], dst, sem).wait()

            gather(0, 0, gs0)

            def pair_body(c2, carry2):
                c0 = 2 * c2

                @pl.when(c2 > 0)
                def _():
                    write_wait(c0 - 1, 1, ws1)

                gather(c0 + 1, 1, gs1)
                gather_wait(c0, 0, gs0)
                add_bias(0)
                write(c0, 0, ws0)
                gather_wait(c0 + 1, 1, gs1)
                add_bias(1)
                write_wait(c0, 0, ws0)

                @pl.when(c2 < NCHUNK // 2 - 1)
                def _():
                    gather(c0 + 2, 0, gs0)

                write(c0 + 1, 1, ws1)
                return carry2

            lax.fori_loop(0, NCHUNK // 2, pair_body, 0)
            write_wait(NCHUNK - 1, 1, ws1)
            return carry

        lax.fori_loop(0, ROWS_PER_W, row_body, 0)

    return k


_sc_call = _make_sc_kernel()


def kernel(freq_positions, time_positions, freq_relative_emb, time_relative_emb, W_dist, b_dist):
    fp = freq_positions.reshape(-1).astype(jnp.int32)
    tp = time_positions.reshape(-1).astype(jnp.int32)
    vf = jnp.arange(NFI, dtype=jnp.float32) - (NF - 1)
    vt = jnp.arange(NTI, dtype=jnp.float32) - (NT - 1)
    dist = (vf[:, None, None] * W_dist[0][None, None, :]
            + vt[None, :, None] * W_dist[1][None, None, :]
            + b_dist[None, None, :])
    fpart = jnp.broadcast_to(freq_relative_emb[:, None, :], (NFI, NTI, D4))
    tpart = jnp.broadcast_to(time_relative_emb[None, :, :], (NFI, NTI, D4))
    full = jnp.concatenate([dist, fpart, tpart], axis=-1).reshape(NFI * NTI, EMBED)
    wflat = jnp.concatenate([W_dist[0], W_dist[1]])
    out = _sc_call(fp, tp, full, wflat)
    return out.reshape(B, S, EMBED)
